# baseline (speedup 1.0000x reference)
import functools

import jax
import jax.numpy as jnp
from jax import lax
from jax.experimental import pallas as pl
from jax.experimental.pallas import tpu as pltpu
from jax.experimental.pallas import tpu_sc as plsc

B = 16384
K = 64
DIM = 100000
NOUT = 256
NC = 2
NS = 16
NW = NC * NS
FPW = K // NW
HD = 49920
HLEN = (HD, DIM - HD)
ICH = 4096
NICH = B // ICH
LANES = 16


def _sc_gather_prod_kernel(i0_hbm, i1_hbm, i2_hbm, t0_hbm, t1_hbm, t2_hbm,
                           out_hbm, rowa_v, rowb_v, acc_v, ixa_v, ixb_v,
                           sr0, sr1, si0, si1):
    wid = lax.axis_index("s") * NC + lax.axis_index("c")
    tabs = (t0_hbm, t1_hbm, t2_hbm)
    idxs = (i0_hbm, i1_hbm, i2_hbm)
    rows = (rowa_v, rowb_v)
    rsems = (sr0, sr1)
    ixbufs = (ixa_v, ixb_v)
    isems = (si0, si1)
    passes = [(f, t, h) for f in range(FPW) for t in range(3)
              for h in range(2)]

    def start_row(p):
        f, t, h = passes[p]
        k = wid * FPW + f
        return pltpu.async_copy(tabs[t].at[k, pl.ds(h * HD, HLEN[h])],
                                rows[p % 2].at[pl.ds(0, HLEN[h])],
                                rsems[p % 2])

    cps = {0: start_row(0)}
    for p, (f, t, h) in enumerate(passes):
        cps.pop(p).wait()
        if p + 1 < len(passes):
            cps[p + 1] = start_row(p + 1)
        rbuf = rows[p % 2]
        lo = h * HD
        first = (t == 0 and h == 0)

        icp = {0: pltpu.async_copy(idxs[t].at[pl.ds(0, ICH)], ixbufs[0],
                                   isems[0])}
        for ci in range(NICH):
            icp.pop(ci).wait()
            if ci + 1 < NICH:
                icp[ci + 1] = pltpu.async_copy(
                    idxs[t].at[pl.ds((ci + 1) * ICH, ICH)],
                    ixbufs[(ci + 1) % 2], isems[(ci + 1) % 2])
            ixb = ixbufs[ci % 2]

            @plsc.parallel_loop(0, ICH // LANES, unroll=8)
            def vloop(v, _ci=ci, _lo=lo, _first=first, _ixb=ixb, _rbuf=rbuf,
                      _h=h):
                iv = _ixb[pl.ds(v * LANES, LANES)]
                local = iv - _lo
                m = plsc.bitcast(local, jnp.uint32) < jnp.uint32(HLEN[_h])
                safe = jnp.where(m, local, 0)
                g = plsc.load_gather(_rbuf, [safe], mask=m)
                val = jnp.where(m, g, jnp.float32(1.0))
                off = _ci * ICH + v * LANES
                if _first:
                    acc_v[pl.ds(off, LANES)] = val
                else:
                    acc_v[pl.ds(off, LANES)] = acc_v[pl.ds(off, LANES)] * val

        if t == 2 and h == 1:
            k = wid * FPW + f
            pltpu.sync_copy(acc_v, out_hbm.at[k, pl.ds(0, B)])


@jax.jit
def _sc_gather_prod(i0, i1, i2, t0t, t1t, t2t):
    mesh = plsc.VectorSubcoreMesh(core_axis_name="c", subcore_axis_name="s")
    return pl.kernel(
        _sc_gather_prod_kernel,
        mesh=mesh,
        compiler_params=pltpu.CompilerParams(use_tc_tiling_on_sc=True,
                                             needs_layout_passes=False),
        out_type=jax.ShapeDtypeStruct((K, B), jnp.float32),
        scratch_types=[
            pltpu.VMEM((DIM - HD,), jnp.float32),
            pltpu.VMEM((DIM - HD,), jnp.float32),
            pltpu.VMEM((B,), jnp.float32),
            pltpu.VMEM((ICH,), jnp.int32),
            pltpu.VMEM((ICH,), jnp.int32),
            pltpu.SemaphoreType.DMA,
            pltpu.SemaphoreType.DMA,
            pltpu.SemaphoreType.DMA,
            pltpu.SemaphoreType.DMA,
        ],
    )(i0, i1, i2, t0t, t1t, t2t)


BM = 2048


def _tc_proj_kernel(prodt_ref, f3_ref, ls_ref, out_ref, ls_out_ref):
    out_ref[...] = lax.dot_general(
        prodt_ref[...], f3_ref[...],
        dimension_numbers=(((0,), (1,)), ((), ())),
        preferred_element_type=jnp.float32,
    )
    ls_out_ref[...] = jnp.clip(ls_ref[...], -2.5, 0.0)


@jax.jit
def _tc_proj(prodt, f3, log_sigma):
    return pl.pallas_call(
        _tc_proj_kernel,
        grid=(B // BM,),
        in_specs=[
            pl.BlockSpec((K, BM), lambda i: (0, i)),
            pl.BlockSpec((NOUT, K), lambda i: (0, 0)),
            pl.BlockSpec((1, NOUT), lambda i: (0, 0)),
        ],
        out_specs=[
            pl.BlockSpec((BM, NOUT), lambda i: (i, 0)),
            pl.BlockSpec((1, NOUT), lambda i: (0, 0)),
        ],
        out_shape=[
            jax.ShapeDtypeStruct((B, NOUT), jnp.float32),
            jax.ShapeDtypeStruct((1, NOUT), jnp.float32),
        ],
    )(prodt, f3, log_sigma)


def kernel(indices, f0, f1, f2, f3, log_sigma):
    idx = indices.astype(jnp.int32)
    prodt = _sc_gather_prod(idx[:, 0], idx[:, 1], idx[:, 2],
                            f0.T, f1.T, f2.T)
    res, ls = _tc_proj(prodt, f3, log_sigma)
    return (res, ls)


# speedup vs baseline: 1.2001x; 1.2001x over previous
import functools

import jax
import jax.numpy as jnp
from jax import lax
from jax.experimental import pallas as pl
from jax.experimental.pallas import tpu as pltpu
from jax.experimental.pallas import tpu_sc as plsc

B = 16384
K = 64
DIM = 100000
NOUT = 256
NC = 2
NS = 16
NW = NC * NS
FPW = K // NW
ICH = 4096
NICH = B // ICH
LANES = 16


def _sc_gather_prod_kernel(i0_hbm, i1_hbm, i2_hbm, t0_hbm, t1_hbm, t2_hbm,
                           out_hbm, row_v, acc_v, ixa_v, ixb_v, sdma,
                           si0, si1):
    wid = lax.axis_index("s") * NC + lax.axis_index("c")
    ixbufs = (ixa_v, ixb_v)
    isems = (si0, si1)

    for f in range(FPW):
        k = wid * FPW + f
        for t, t_hbm, i_hbm in ((0, t0_hbm, i0_hbm), (1, t1_hbm, i1_hbm),
                                (2, t2_hbm, i2_hbm)):
            rcp = pltpu.async_copy(t_hbm.at[k, pl.ds(0, DIM)], row_v, sdma)
            icp = {0: pltpu.async_copy(i_hbm.at[pl.ds(0, ICH)], ixbufs[0],
                                       isems[0])}
            rcp.wait()
            for ci in range(NICH):
                icp.pop(ci).wait()
                if ci + 1 < NICH:
                    icp[ci + 1] = pltpu.async_copy(
                        i_hbm.at[pl.ds((ci + 1) * ICH, ICH)],
                        ixbufs[(ci + 1) % 2], isems[(ci + 1) % 2])
                ixb = ixbufs[ci % 2]

                @plsc.parallel_loop(0, ICH // LANES, unroll=8)
                def vloop(v, _t=t, _ci=ci, _ixb=ixb):
                    iv = _ixb[pl.ds(v * LANES, LANES)]
                    g = plsc.load_gather(row_v, [iv])
                    off = _ci * ICH + v * LANES
                    if _t == 0:
                        acc_v[pl.ds(off, LANES)] = g
                    else:
                        acc_v[pl.ds(off, LANES)] = acc_v[pl.ds(off, LANES)] * g
        pltpu.sync_copy(acc_v, out_hbm.at[k, pl.ds(0, B)])


@jax.jit
def _sc_gather_prod(i0, i1, i2, t0t, t1t, t2t):
    mesh = plsc.VectorSubcoreMesh(core_axis_name="c", subcore_axis_name="s")
    return pl.kernel(
        _sc_gather_prod_kernel,
        mesh=mesh,
        compiler_params=pltpu.CompilerParams(use_tc_tiling_on_sc=True,
                                             needs_layout_passes=False),
        out_type=jax.ShapeDtypeStruct((K, B), jnp.float32),
        scratch_types=[
            pltpu.VMEM((DIM,), jnp.float32),
            pltpu.VMEM((B,), jnp.float32),
            pltpu.VMEM((ICH,), jnp.int32),
            pltpu.VMEM((ICH,), jnp.int32),
            pltpu.SemaphoreType.DMA,
            pltpu.SemaphoreType.DMA,
            pltpu.SemaphoreType.DMA,
        ],
    )(i0, i1, i2, t0t, t1t, t2t)


BM = 2048


def _tc_proj_kernel(prodt_ref, f3_ref, ls_ref, out_ref, ls_out_ref):
    out_ref[...] = lax.dot_general(
        prodt_ref[...], f3_ref[...],
        dimension_numbers=(((0,), (1,)), ((), ())),
        preferred_element_type=jnp.float32,
    )
    ls_out_ref[...] = jnp.clip(ls_ref[...], -2.5, 0.0)


@jax.jit
def _tc_proj(prodt, f3, log_sigma):
    return pl.pallas_call(
        _tc_proj_kernel,
        grid=(B // BM,),
        in_specs=[
            pl.BlockSpec((K, BM), lambda i: (0, i)),
            pl.BlockSpec((NOUT, K), lambda i: (0, 0)),
            pl.BlockSpec((1, NOUT), lambda i: (0, 0)),
        ],
        out_specs=[
            pl.BlockSpec((BM, NOUT), lambda i: (i, 0)),
            pl.BlockSpec((1, NOUT), lambda i: (0, 0)),
        ],
        out_shape=[
            jax.ShapeDtypeStruct((B, NOUT), jnp.float32),
            jax.ShapeDtypeStruct((1, NOUT), jnp.float32),
        ],
    )(prodt, f3, log_sigma)


def kernel(indices, f0, f1, f2, f3, log_sigma):
    idx = indices.astype(jnp.int32)
    prodt = _sc_gather_prod(idx[:, 0], idx[:, 1], idx[:, 2],
                            f0.T, f1.T, f2.T)
    res, ls = _tc_proj(prodt, f3, log_sigma)
    return (res, ls)
